# baseline (speedup 1.0000x reference)
import functools

import jax
import jax.numpy as jnp
from jax import lax
from jax.experimental import pallas as pl
from jax.experimental.pallas import tpu as pltpu
from jax.experimental.pallas import tpu_sc as plsc

D = 64
B = 4096
L = 200
HID = 300
NCLS = 100

NC = 2
NS = 16
NW = NC * NS
ROWS_PER = B // NW
CHUNK = 40
CPR = L // CHUNK
NCHUNKS = ROWS_PER * CPR


def _pool_body(x_hbm, table_hbm, out_hbm, idx_v, rows_v, acc_v, sems):
    wid = lax.axis_index("s") * NC + lax.axis_index("c")
    pltpu.sync_copy(x_hbm.at[wid], idx_v)

    def gather(j, p):
        return pltpu.make_async_copy(
            table_hbm.at[idx_v.at[j]], rows_v.at[p], sems.at[p])

    gather(0, 0).start()

    def row_body(r, carry):
        accs = (jnp.zeros((16,), jnp.float32),) * 4
        for c in range(CPR):
            j = r * CPR + c
            p = lax.rem(j, 2)
            gather(j, p).wait()

            @pl.when(j + 1 < NCHUNKS)
            def _():
                gather(j + 1, 1 - p).start()

            def tacc(t, a):
                t0 = t * 8
                out = list(a)
                for u in range(8):
                    for k in range(4):
                        out[k] = out[k] + rows_v[p, t0 + u, pl.ds(k * 16, 16)]
                return tuple(out)

            accs = lax.fori_loop(0, CHUNK // 8, tacc, accs)
        for k in range(4):
            acc_v[r, pl.ds(k * 16, 16)] = accs[k]
        return carry

    lax.fori_loop(0, ROWS_PER, row_body, 0)
    pltpu.sync_copy(acc_v, out_hbm.at[wid])


_pool = functools.partial(
    pl.kernel,
    out_type=jax.ShapeDtypeStruct((NW, ROWS_PER, D), jnp.float32),
    mesh=plsc.VectorSubcoreMesh(core_axis_name="c", subcore_axis_name="s"),
    compiler_params=pltpu.CompilerParams(use_tc_tiling_on_sc=False),
    scratch_types=[
        pltpu.VMEM((NCHUNKS, CHUNK), jnp.int32),
        pltpu.VMEM((2, CHUNK, D), jnp.float32),
        pltpu.VMEM((ROWS_PER, D), jnp.float32),
        pltpu.SemaphoreType.DMA((2,)),
    ],
)(_pool_body)


def _mlp_body(p_ref, w1_ref, b1_ref, w2_ref, b2_ref, o_ref):
    h = jnp.dot(p_ref[...] * (1.0 / L), w1_ref[...],
                preferred_element_type=jnp.float32) + b1_ref[...]
    h = jnp.maximum(h, 0.0)
    o_ref[...] = jnp.dot(h, w2_ref[...],
                         preferred_element_type=jnp.float32) + b2_ref[...]


def _mlp(pooled, W1, b1, W2, b2):
    return pl.pallas_call(
        _mlp_body,
        out_shape=jax.ShapeDtypeStruct((B, NCLS), jnp.float32),
    )(pooled, W1, b1.reshape(1, HID), W2, b2.reshape(1, NCLS))


def kernel(x, table, W1, b1, W2, b2):
    x3 = x.astype(jnp.int32).reshape(NW, NCHUNKS, CHUNK)
    pooled = _pool(x3, table)
    return _mlp(pooled.reshape(B, D), W1, b1, W2, b2)
